# Initial kernel scaffold; baseline (speedup 1.0000x reference)
import functools

import jax
import jax.numpy as jnp
from jax import lax
from jax.experimental import pallas as pl
from jax.experimental.pallas import tpu as pltpu
from jax.experimental.pallas import tpu_sc as plsc

BATCH = 4096
SEQ = 50
DIM = 128
LANES = 16
NCORE = 2
NSUB = 16
SENT_PER_W = BATCH // (NCORE * NSUB)
CHUNK_SENT = 2
CHUNK_ROWS = CHUNK_SENT * SEQ
NCHUNK = SENT_PER_W // CHUNK_SENT
LGROUPS = DIM // LANES


def _embed_body(idx_hbm, table_hbm, out_hbm, idx_v, rows_v, out_v, sem0, sem1):
    c = lax.axis_index("c")
    s = lax.axis_index("s")
    sent_base = c * (NSUB * SENT_PER_W) + s * SENT_PER_W
    chunk_base = sent_base // CHUNK_SENT

    pltpu.sync_copy(idx_hbm.at[pl.ds(chunk_base, NCHUNK)], idx_v)

    scale = jnp.float32(1.0 / SEQ)

    def gather(p, buf, sem):
        return pltpu.async_copy(table_hbm.at[idx_v.at[p]], rows_v.at[buf], sem)

    def wait(p, buf, sem):
        pltpu.make_async_copy(table_hbm.at[idx_v.at[p]], rows_v.at[buf], sem).wait()

    def reduce_sentence(buf, base_row):
        def body(t, accs):
            r0 = base_row + 2 * t
            return tuple(
                accs[l] + rows_v[buf, r0, pl.ds(l * LANES, LANES)]
                + rows_v[buf, r0 + 1, pl.ds(l * LANES, LANES)]
                for l in range(LGROUPS)
            )
        init = tuple(jnp.zeros((LANES,), jnp.float32) for _ in range(LGROUPS))
        return lax.fori_loop(0, SEQ // 2, body, init)

    def compute(p, buf):
        for s2 in range(CHUNK_SENT):
            accs = reduce_sentence(buf, s2 * SEQ)
            for l in range(LGROUPS):
                out_v[CHUNK_SENT * p + s2, pl.ds(l * LANES, LANES)] = accs[l] * scale

    gather(0, 0, sem0)

    def outer(q, carry):
        p0 = 2 * q
        gather(p0 + 1, 1, sem1)
        wait(p0, 0, sem0)
        compute(p0, 0)

        @pl.when(q < NCHUNK // 2 - 1)
        def _():
            gather(p0 + 2, 0, sem0)

        wait(p0 + 1, 1, sem1)
        compute(p0 + 1, 1)
        return carry

    lax.fori_loop(0, NCHUNK // 2, outer, 0)

    pltpu.sync_copy(out_v, out_hbm.at[pl.ds(sent_base, SENT_PER_W)])


_embed = functools.partial(
    pl.kernel,
    mesh=plsc.VectorSubcoreMesh(core_axis_name="c", subcore_axis_name="s"),
    out_type=jax.ShapeDtypeStruct((BATCH, DIM), jnp.float32),
    scratch_types=[
        pltpu.VMEM((BATCH * SEQ // CHUNK_ROWS, CHUNK_ROWS), jnp.int32),
        pltpu.VMEM((2, CHUNK_ROWS, DIM), jnp.float32),
        pltpu.VMEM((SENT_PER_W, DIM), jnp.float32),
        pltpu.SemaphoreType.DMA,
        pltpu.SemaphoreType.DMA,
    ],
)(_embed_body)


def kernel(indices, table):
    idx2 = indices.astype(jnp.int32).reshape(BATCH * SEQ // CHUNK_ROWS, CHUNK_ROWS)
    return _embed(idx2, table)


# speedup vs baseline: 4.3187x; 4.3187x over previous
import functools

import jax
import jax.numpy as jnp
from jax import lax
from jax.experimental import pallas as pl
from jax.experimental.pallas import tpu as pltpu
from jax.experimental.pallas import tpu_sc as plsc

BATCH = 4096
SEQ = 50
DIM = 128
LANES = 16
NCORE = 2
NSUB = 16
SENT_PER_W = BATCH // (NCORE * NSUB)
CHUNK_SENT = 2
CHUNK_ROWS = CHUNK_SENT * SEQ
NCHUNK = SENT_PER_W // CHUNK_SENT
LGROUPS = DIM // LANES


def _embed_body(idx_hbm, table_hbm, out_hbm, idx_v, rows_v, out_v, sem0, sem1):
    c = lax.axis_index("c")
    s = lax.axis_index("s")
    sent_base = pl.multiple_of(c * (NSUB * SENT_PER_W) + s * SENT_PER_W, SENT_PER_W)
    chunk_base = pl.multiple_of(sent_base // CHUNK_SENT, NCHUNK)

    pltpu.sync_copy(idx_hbm.at[pl.ds(chunk_base, NCHUNK)], idx_v)

    scale = jnp.float32(1.0 / SEQ)

    def gather(p, buf, sem):
        return pltpu.async_copy(table_hbm.at[idx_v.at[p]], rows_v.at[buf], sem)

    def wait(p, buf, sem):
        pltpu.make_async_copy(table_hbm.at[idx_v.at[p]], rows_v.at[buf], sem).wait()

    def reduce_sentence(buf, base_row):
        def body(t, accs):
            r0 = base_row + 2 * t
            return tuple(
                accs[l] + rows_v[buf, r0, pl.ds(l * LANES, LANES)]
                + rows_v[buf, r0 + 1, pl.ds(l * LANES, LANES)]
                for l in range(LGROUPS)
            )
        init = tuple(jnp.zeros((LANES,), jnp.float32) for _ in range(LGROUPS))
        return lax.fori_loop(0, SEQ // 2, body, init)

    def compute(p, buf):
        for s2 in range(CHUNK_SENT):
            accs = reduce_sentence(buf, s2 * SEQ)
            for l in range(LGROUPS):
                out_v[CHUNK_SENT * p + s2, pl.ds(l * LANES, LANES)] = accs[l] * scale

    gather(0, 0, sem0)

    def outer(q, carry):
        p0 = 2 * q
        gather(p0 + 1, 1, sem1)
        wait(p0, 0, sem0)
        compute(p0, 0)

        @pl.when(q < NCHUNK // 2 - 1)
        def _():
            gather(p0 + 2, 0, sem0)

        wait(p0 + 1, 1, sem1)
        compute(p0 + 1, 1)
        return carry

    lax.fori_loop(0, NCHUNK // 2, outer, 0)

    pltpu.sync_copy(out_v, out_hbm.at[pl.ds(sent_base, SENT_PER_W)])


_embed = functools.partial(
    pl.kernel,
    mesh=plsc.VectorSubcoreMesh(core_axis_name="c", subcore_axis_name="s"),
    out_type=jax.ShapeDtypeStruct((BATCH, DIM), jnp.float32),
    scratch_types=[
        pltpu.VMEM((NCHUNK, CHUNK_ROWS), jnp.int32),
        pltpu.VMEM((2, CHUNK_ROWS, DIM), jnp.float32),
        pltpu.VMEM((SENT_PER_W, DIM), jnp.float32),
        pltpu.SemaphoreType.DMA,
        pltpu.SemaphoreType.DMA,
    ],
)(_embed_body)


def kernel(indices, table):
    idx2 = indices.astype(jnp.int32).reshape(BATCH * SEQ // CHUNK_ROWS, CHUNK_ROWS)
    return _embed(idx2, table)
